# baseline (speedup 1.0000x reference)
import jax
import jax.numpy as jnp
from jax import lax
from jax.experimental import pallas as pl
from jax.experimental.pallas import tpu as pltpu
from jax.experimental.pallas import tpu_sc as plsc

B, D, V = 16384, 128, 1000000

NC = 2
NS = 16
NL = 16
NW = NC * NS
BPW = B // NW
GC = 128
KG = BPW // GC
RC = 64
KR = BPW // RC
NBUF = 4
NACC = 4
_INV = 1.0 / float(B * D)


def _sc_body(idx_hbm, table_hbm, l_hbm, r_hbm, out_hbm,
             idx_v, alpha_v, lbuf, rbuf, acc_v, sem_a, sem_d):
    wid = lax.axis_index("s") * NC + lax.axis_index("c")
    base = wid * BPW

    pltpu.sync_copy(idx_hbm.at[pl.ds(wid * KG, KG)], idx_v)
    a_cps = [
        pltpu.async_copy(table_hbm.at[idx_v.at[j]],
                         alpha_v.at[pl.ds(j * GC, GC)], sem_a)
        for j in range(KG)
    ]

    for c in range(NBUF):
        pltpu.async_copy(l_hbm.at[pl.ds(base + c * RC, RC), :], lbuf.at[c],
                         sem_d.at[c])
        pltpu.async_copy(r_hbm.at[pl.ds(base + c * RC, RC), :], rbuf.at[c],
                         sem_d.at[c])
    for cp in a_cps:
        cp.wait()

    def chunk_body(c, accs_t):
        b = lax.rem(c, NBUF)
        pltpu.make_async_copy(l_hbm.at[pl.ds(0, RC), :], lbuf.at[b],
                              sem_d.at[b]).wait()
        pltpu.make_async_copy(r_hbm.at[pl.ds(0, RC), :], rbuf.at[b],
                              sem_d.at[b]).wait()

        def group_body(g, at):
            a16 = alpha_v[pl.ds(pl.multiple_of(c * RC + g * NL, NL), NL)]
            al = list(at)
            for j in range(NL):
                a_s = a16[j]
                r = g * NL + j
                for gg in range(D // NL):
                    lv = lbuf[b, r, pl.ds(gg * NL, NL)]
                    rv = rbuf[b, r, pl.ds(gg * NL, NL)]
                    al[gg % NACC] = al[gg % NACC] + jnp.abs(lv - a_s * rv)
            return tuple(al)

        accs_t = lax.fori_loop(0, RC // NL, group_body, accs_t)

        cn = c + NBUF

        @pl.when(cn < KR)
        def _fire_next():
            pltpu.async_copy(l_hbm.at[pl.ds(base + cn * RC, RC), :],
                             lbuf.at[b], sem_d.at[b])
            pltpu.async_copy(r_hbm.at[pl.ds(base + cn * RC, RC), :],
                             rbuf.at[b], sem_d.at[b])

        return accs_t

    accs = lax.fori_loop(0, KR, chunk_body,
                         tuple(jnp.zeros((NL,), jnp.float32)
                               for _ in range(NACC)))

    total = accs[0]
    for gg in range(1, NACC):
        total = total + accs[gg]
    acc_v[...] = total
    pltpu.sync_copy(acc_v, out_hbm.at[wid])


_sc_loss = pl.kernel(
    _sc_body,
    mesh=plsc.VectorSubcoreMesh(core_axis_name="c", subcore_axis_name="s"),
    out_type=jax.ShapeDtypeStruct((NW, NL), jnp.float32),
    scratch_types=[
        pltpu.VMEM((KG, GC), jnp.int32),
        pltpu.VMEM((BPW,), jnp.float32),
        pltpu.VMEM((NBUF, RC, D), jnp.float32),
        pltpu.VMEM((NBUF, RC, D), jnp.float32),
        pltpu.VMEM((NL,), jnp.float32),
        pltpu.SemaphoreType.DMA,
        pltpu.SemaphoreType.DMA((NBUF,)),
    ],
)


def _fin_body(p_ref, out_ref):
    out_ref[0, 0] = jnp.sum(p_ref[...]) * _INV


_finish = pl.pallas_call(
    _fin_body,
    out_specs=pl.BlockSpec(memory_space=pltpu.SMEM),
    out_shape=jax.ShapeDtypeStruct((1, 1), jnp.float32),
)


def kernel(emd_l, emd_r, player, alpha_weight):
    idx = player.astype(jnp.int32).reshape(NW * KG, GC)
    table = alpha_weight.reshape(V)
    parts = _sc_loss(idx, table, emd_l, emd_r)
    return _finish(parts)[0, 0]
